# Initial kernel scaffold; baseline (speedup 1.0000x reference)
import jax
import jax.numpy as jnp
from jax.experimental import pallas as pl


def kernel(x, base_deformation, base_stride, base_beta_fwd, base_beta_bwd, base_strength, base_alpha_fwd, base_alpha_bwd, sample_bias):
    raise NotImplementedError("write your pallas kernel here")


# speedup vs baseline: 1803.8659x; 1803.8659x over previous
import functools

import jax
import jax.numpy as jnp
from jax import lax
from jax.experimental import pallas as pl
from jax.experimental.pallas import tpu as pltpu
from jax.experimental.pallas import tpu_sc as plsc

_K = 16
_LANES = 16
_NC = 2
_NS = 16
_PAD = 256


def _sc_scatter_body(L, n_chunks, d_hbm, w_hbm, rw_hbm, zeros_hbm,
                     out_hbm, d_v, w_v, rw_v, acc_v):
    Lp = L + _PAD
    cid = lax.axis_index("c")
    sid = lax.axis_index("s")
    wid = cid * _NS + sid

    pltpu.sync_copy(d_hbm, d_v)
    pltpu.sync_copy(w_hbm, w_v)
    pltpu.sync_copy(rw_hbm, rw_v)
    pltpu.sync_copy(zeros_hbm, acc_v)

    iota_i = jax.lax.iota(jnp.int32, _LANES)
    iota_f = iota_i.astype(jnp.float32)
    m_even = (iota_i & 1) == 0
    m_odd = jnp.logical_not(m_even)
    zero16 = jnp.zeros((_LANES,), jnp.int32)
    one16 = jnp.ones((_LANES,), jnp.int32)
    dvec = d_v[...]

    l_base = (wid * (L // (_NC * _NS))).astype(jnp.float32)
    fmax = float(L - 1)

    carry0 = (jnp.zeros((_LANES,), jnp.float32),) * 4

    for k in range(_K):
        wk = w_v[k]
        rwk = rw_v[k]

        def body(j, c, wk=wk, rwk=rwk):
            blo_w, blo_h, bhi_w, bhi_h = c
            lvec = l_base + (j * _LANES).astype(jnp.float32) + iota_f
            centers = lvec + dvec
            p = centers + wk
            p_cl = jnp.minimum(jnp.maximum(p, 0.0), fmax)
            pf_i = p_cl.astype(jnp.int32)
            frac = p_cl - pf_i.astype(jnp.float32)
            wfl = (1.0 - frac) * rwk
            wcl = frac * rwk
            m_in = jnp.logical_and(p > 0.0, p < fmax)
            pc_i = pf_i + 1
            for m_par in (m_even, m_odd):
                mm = jnp.logical_and(m_in, m_par)
                plsc.addupdate_scatter(acc_v, [zero16, pf_i], wfl, mask=mm)
                plsc.addupdate_scatter(acc_v, [zero16, pc_i], wcl, mask=mm)
                plsc.addupdate_scatter(acc_v, [one16, pf_i], 1.0 - frac, mask=mm)
                plsc.addupdate_scatter(acc_v, [one16, pc_i], frac, mask=mm)
            m_lo = p <= 0.0
            m_hi = p >= fmax
            blo_w = blo_w + jnp.where(m_lo, rwk, 0.0)
            blo_h = blo_h + jnp.where(m_lo, 1.0, 0.0)
            bhi_w = bhi_w + jnp.where(m_hi, rwk, 0.0)
            bhi_h = bhi_h + jnp.where(m_hi, 1.0, 0.0)
            return (blo_w, blo_h, bhi_w, bhi_h)

        carry0 = lax.fori_loop(0, n_chunks, body, carry0)

    blo_w, blo_h, bhi_w, bhi_h = carry0
    col_lo = iota_i + L
    col_hi = iota_i + (L + 128)
    plsc.addupdate_scatter(acc_v, [zero16, col_lo], blo_w)
    plsc.addupdate_scatter(acc_v, [one16, col_lo], blo_h)
    plsc.addupdate_scatter(acc_v, [zero16, col_hi], bhi_w)
    plsc.addupdate_scatter(acc_v, [one16, col_hi], bhi_h)

    pltpu.sync_copy(acc_v, out_hbm.at[wid])


def _sc_scatter(L, d16, w16, rw16):
    Lp = L + _PAD
    n_chunks = L // (_NC * _NS * _LANES)
    mesh = plsc.VectorSubcoreMesh(core_axis_name="c", subcore_axis_name="s")
    zeros = jnp.zeros((2, Lp), jnp.float32)
    fn = functools.partial(
        pl.kernel,
        out_type=jax.ShapeDtypeStruct((_NC * _NS, 2, Lp), jnp.float32),
        mesh=mesh,
        compiler_params=pltpu.CompilerParams(needs_layout_passes=False),
        scratch_types=[
            pltpu.VMEM((_LANES,), jnp.float32),
            pltpu.VMEM((_K, _LANES), jnp.float32),
            pltpu.VMEM((_K, _LANES), jnp.float32),
            pltpu.VMEM((2, Lp), jnp.float32),
        ],
    )(functools.partial(_sc_scatter_body, L, n_chunks))
    return fn(d16, w16, rw16, zeros)


def _tc_norm_body(L, parts_ref, norm_ref):
    s = jnp.sum(parts_ref[...], axis=0)
    pw = s[0:1, :]
    ph = s[1:2, :]
    b0w = jnp.sum(pw[:, L:L + 16])
    b0h = jnp.sum(ph[:, L:L + 16])
    bLw = jnp.sum(pw[:, L + 128:L + 144])
    bLh = jnp.sum(ph[:, L + 128:L + 144])
    col = lax.broadcasted_iota(jnp.int32, (1, L), 1)
    wm = pw[:, :L] + jnp.where(col == 0, b0w, 0.0) + jnp.where(col == L - 1, bLw, 0.0)
    hc = ph[:, :L] + jnp.where(col == 0, b0h, 0.0) + jnp.where(col == L - 1, bLh, 0.0)
    avg = wm / jnp.maximum(hc, 1e-6)
    norm_ref[...] = avg / jnp.maximum(jnp.sum(avg, axis=1, keepdims=True), 1e-6)


def _tc_norm(L, parts):
    return pl.pallas_call(
        functools.partial(_tc_norm_body, L),
        out_shape=jax.ShapeDtypeStruct((1, L), jnp.float32),
    )(parts)


def _tc_mul_body(norm_ref, x_ref, o_ref):
    o_ref[...] = x_ref[...] * norm_ref[...]


def _tc_mul(x, norm):
    B, L = x.shape
    rows = 16
    grid = (B // rows,)
    return pl.pallas_call(
        _tc_mul_body,
        grid=grid,
        in_specs=[
            pl.BlockSpec((1, L), lambda i: (0, 0)),
            pl.BlockSpec((rows, L), lambda i: (i, 0)),
        ],
        out_specs=pl.BlockSpec((rows, L), lambda i: (i, 0)),
        out_shape=jax.ShapeDtypeStruct((B, L), x.dtype),
    )(norm, x)


def kernel(x, base_deformation, base_stride, base_beta_fwd, base_beta_bwd,
           base_strength, base_alpha_fwd, base_alpha_bwd, sample_bias):
    B, L = x.shape
    K = _K
    d = jnp.clip(base_deformation, -32.0, 32.0)
    stride = jax.nn.softplus(base_stride)
    beta_fwd = jax.nn.softplus(base_beta_fwd)
    beta_bwd = jax.nn.softplus(base_beta_bwd)
    strength = jax.nn.softplus(base_strength)
    alpha_fwd = jax.nn.softplus(base_alpha_fwd)
    alpha_bwd = jax.nn.softplus(base_alpha_bwd)
    k = jnp.arange(K, dtype=jnp.float32) - (K // 2)
    k_abs = jnp.abs(k)
    warped = jnp.where(k >= 0, (k_abs ** beta_fwd) * stride,
                       -((k_abs ** beta_bwd) * stride))
    envelope = jnp.where(k >= 0, strength / (1.0 + k_abs) ** alpha_fwd,
                         strength / (1.0 + k_abs) ** alpha_bwd)
    rw = envelope * (1.0 + jnp.tanh(sample_bias))

    d16 = jnp.broadcast_to(d, (_LANES,))
    w16 = jnp.broadcast_to(warped[:, None], (K, _LANES))
    rw16 = jnp.broadcast_to(rw[:, None], (K, _LANES))

    parts = _sc_scatter(L, d16, w16, rw16)
    norm = _tc_norm(L, parts)
    return _tc_mul(x, norm)


# speedup vs baseline: 1898.0566x; 1.0522x over previous
import functools

import jax
import jax.numpy as jnp
from jax import lax
from jax.experimental import pallas as pl
from jax.experimental.pallas import tpu as pltpu
from jax.experimental.pallas import tpu_sc as plsc

_K = 16
_LANES = 16
_NC = 2
_NS = 16
_PAD = 256


def _sc_scatter_body(L, n_chunks, d_hbm, w_hbm, rw_hbm, zeros_hbm,
                     out_hbm, d_v, w_v, rw_v, acc_v):
    Lp = L + _PAD
    cid = lax.axis_index("c")
    sid = lax.axis_index("s")
    wid = cid * _NS + sid

    pltpu.sync_copy(d_hbm, d_v)
    pltpu.sync_copy(w_hbm, w_v)
    pltpu.sync_copy(rw_hbm, rw_v)
    pltpu.sync_copy(zeros_hbm, acc_v)

    iota_i = jax.lax.iota(jnp.int32, _LANES)
    iota_f = iota_i.astype(jnp.float32)
    zero16 = jnp.zeros((_LANES,), jnp.int32)
    one16 = jnp.ones((_LANES,), jnp.int32)
    zf = jnp.zeros((_LANES,), jnp.float32)
    dvec = d_v[...]

    l_base = (wid * (L // (_NC * _NS))).astype(jnp.float32)
    lvec0 = l_base + iota_f
    fmax = float(L - 1)

    blo_w = blo_h = bhi_w = bhi_h = zf

    for k in range(_K):
        wk = w_v[k]
        rwk = rw_v[k]

        def body(j, c, wk=wk, rwk=rwk):
            lvec, c_lo, c_hi = c
            centers = lvec + dvec
            p = centers + wk
            p_cl = jnp.minimum(jnp.maximum(p, 0.0), fmax)
            pf_i = p_cl.astype(jnp.int32)
            frac = p_cl - pf_i.astype(jnp.float32)
            omf = 1.0 - frac
            m_lo = p <= 0.0
            m_hi = p >= fmax
            m_in = jnp.logical_not(jnp.logical_or(m_lo, m_hi))
            pc_i = pf_i + 1
            plsc.addupdate_scatter(acc_v, [zero16, pf_i], omf * rwk, mask=m_in)
            plsc.addupdate_scatter(acc_v, [zero16, pc_i], frac * rwk, mask=m_in)
            plsc.addupdate_scatter(acc_v, [one16, pf_i], omf, mask=m_in)
            plsc.addupdate_scatter(acc_v, [one16, pc_i], frac, mask=m_in)
            c_lo = c_lo + jnp.where(m_lo, 1.0, 0.0)
            c_hi = c_hi + jnp.where(m_hi, 1.0, 0.0)
            return (lvec + 16.0, c_lo, c_hi)

        _, c_lo, c_hi = lax.fori_loop(0, n_chunks, body, (lvec0, zf, zf))
        blo_w = blo_w + c_lo * rwk
        blo_h = blo_h + c_lo
        bhi_w = bhi_w + c_hi * rwk
        bhi_h = bhi_h + c_hi
    col_lo = iota_i + L
    col_hi = iota_i + (L + 128)
    plsc.addupdate_scatter(acc_v, [zero16, col_lo], blo_w)
    plsc.addupdate_scatter(acc_v, [one16, col_lo], blo_h)
    plsc.addupdate_scatter(acc_v, [zero16, col_hi], bhi_w)
    plsc.addupdate_scatter(acc_v, [one16, col_hi], bhi_h)

    pltpu.sync_copy(acc_v, out_hbm.at[wid])


def _sc_scatter(L, d16, w16, rw16):
    Lp = L + _PAD
    n_chunks = L // (_NC * _NS * _LANES)
    mesh = plsc.VectorSubcoreMesh(core_axis_name="c", subcore_axis_name="s")
    zeros = jnp.zeros((2, Lp), jnp.float32)
    fn = functools.partial(
        pl.kernel,
        out_type=jax.ShapeDtypeStruct((_NC * _NS, 2, Lp), jnp.float32),
        mesh=mesh,
        compiler_params=pltpu.CompilerParams(needs_layout_passes=False),
        scratch_types=[
            pltpu.VMEM((_LANES,), jnp.float32),
            pltpu.VMEM((_K, _LANES), jnp.float32),
            pltpu.VMEM((_K, _LANES), jnp.float32),
            pltpu.VMEM((2, Lp), jnp.float32),
        ],
    )(functools.partial(_sc_scatter_body, L, n_chunks))
    return fn(d16, w16, rw16, zeros)


def _tc_norm_mul_body(L, parts_ref, x_ref, o_ref, norm_ref):
    @pl.when(pl.program_id(0) == 0)
    def _():
        s = jnp.sum(parts_ref[...], axis=0)
        pw = s[0:1, :]
        ph = s[1:2, :]
        b0w = jnp.sum(pw[:, L:L + 16])
        b0h = jnp.sum(ph[:, L:L + 16])
        bLw = jnp.sum(pw[:, L + 128:L + 144])
        bLh = jnp.sum(ph[:, L + 128:L + 144])
        col = lax.broadcasted_iota(jnp.int32, (1, L), 1)
        wm = pw[:, :L] + jnp.where(col == 0, b0w, 0.0) + jnp.where(col == L - 1, bLw, 0.0)
        hc = ph[:, :L] + jnp.where(col == 0, b0h, 0.0) + jnp.where(col == L - 1, bLh, 0.0)
        avg = wm / jnp.maximum(hc, 1e-6)
        norm_ref[...] = avg / jnp.maximum(jnp.sum(avg, axis=1, keepdims=True), 1e-6)

    o_ref[...] = x_ref[...] * norm_ref[...]


def _tc_norm_mul(x, parts):
    B, L = x.shape
    Lp = L + _PAD
    rows = 16
    grid = (B // rows,)
    return pl.pallas_call(
        functools.partial(_tc_norm_mul_body, L),
        grid=grid,
        in_specs=[
            pl.BlockSpec((_NC * _NS, 2, Lp), lambda i: (0, 0, 0)),
            pl.BlockSpec((rows, L), lambda i: (i, 0)),
        ],
        out_specs=pl.BlockSpec((rows, L), lambda i: (i, 0)),
        out_shape=jax.ShapeDtypeStruct((B, L), x.dtype),
        scratch_shapes=[pltpu.VMEM((1, L), jnp.float32)],
    )(parts, x)


def kernel(x, base_deformation, base_stride, base_beta_fwd, base_beta_bwd,
           base_strength, base_alpha_fwd, base_alpha_bwd, sample_bias):
    B, L = x.shape
    K = _K
    d = jnp.clip(base_deformation, -32.0, 32.0)
    stride = jax.nn.softplus(base_stride)
    beta_fwd = jax.nn.softplus(base_beta_fwd)
    beta_bwd = jax.nn.softplus(base_beta_bwd)
    strength = jax.nn.softplus(base_strength)
    alpha_fwd = jax.nn.softplus(base_alpha_fwd)
    alpha_bwd = jax.nn.softplus(base_alpha_bwd)
    k = jnp.arange(K, dtype=jnp.float32) - (K // 2)
    k_abs = jnp.abs(k)
    warped = jnp.where(k >= 0, (k_abs ** beta_fwd) * stride,
                       -((k_abs ** beta_bwd) * stride))
    envelope = jnp.where(k >= 0, strength / (1.0 + k_abs) ** alpha_fwd,
                         strength / (1.0 + k_abs) ** alpha_bwd)
    rw = envelope * (1.0 + jnp.tanh(sample_bias))

    d16 = jnp.broadcast_to(d, (_LANES,))
    w16 = jnp.broadcast_to(warped[:, None], (K, _LANES))
    rw16 = jnp.broadcast_to(rw[:, None], (K, _LANES))

    parts = _sc_scatter(L, d16, w16, rw16)
    return _tc_norm_mul(x, parts)


# speedup vs baseline: 2114.9802x; 1.1143x over previous
import functools

import jax
import jax.numpy as jnp
from jax import lax
from jax.experimental import pallas as pl
from jax.experimental.pallas import tpu as pltpu
from jax.experimental.pallas import tpu_sc as plsc

_K = 16
_LANES = 16
_NC = 2
_NS = 16
_PAD = 256
_UNROLL = 4


def _sc_scatter_body(L, d_hbm, w_hbm, rw_hbm, out_hbm, d_v, w_v, rw_v,
                     wm_v, hc_v):
    n_chunks = L // (_NC * _NS * _LANES)
    cid = lax.axis_index("c")
    sid = lax.axis_index("s")
    wid = cid * _NS + sid

    pltpu.sync_copy(d_hbm, d_v)
    pltpu.sync_copy(w_hbm, w_v)
    pltpu.sync_copy(rw_hbm, rw_v)

    iota_i = jax.lax.iota(jnp.int32, _LANES)
    iota_f = iota_i.astype(jnp.float32)
    zf = jnp.zeros((_LANES,), jnp.float32)
    dvec = d_v[...]

    Lp = L + _PAD

    def zbody(i, _):
        off = i * (4 * _LANES)
        for u in range(4):
            wm_v[pl.ds(off + u * _LANES, _LANES)] = zf
            hc_v[pl.ds(off + u * _LANES, _LANES)] = zf
        return 0

    lax.fori_loop(0, Lp // (4 * _LANES), zbody, 0, unroll=False)

    l_base = (wid * (L // (_NC * _NS))).astype(jnp.float32)
    fmax = float(L - 1)

    blo_w = blo_h = bhi_w = bhi_h = zf

    for k in range(_K):
        wk = w_v[k]
        rwk = rw_v[k]

        def chunk(lvec, c_lo, c_hi, wk=wk, rwk=rwk):
            centers = lvec + dvec
            p = centers + wk
            p_cl = jnp.minimum(jnp.maximum(p, 0.0), fmax)
            pf_i = p_cl.astype(jnp.int32)
            frac = p_cl - pf_i.astype(jnp.float32)
            omf = 1.0 - frac
            m_lo = p <= 0.0
            m_hi = p >= fmax
            m_in = jnp.logical_not(jnp.logical_or(m_lo, m_hi))
            pc_i = pf_i + 1
            plsc.addupdate_scatter(wm_v, [pf_i], omf * rwk, mask=m_in)
            plsc.addupdate_scatter(wm_v, [pc_i], frac * rwk, mask=m_in)
            plsc.addupdate_scatter(hc_v, [pf_i], omf, mask=m_in)
            plsc.addupdate_scatter(hc_v, [pc_i], frac, mask=m_in)
            c_lo = c_lo + jnp.where(m_lo, 1.0, 0.0)
            c_hi = c_hi + jnp.where(m_hi, 1.0, 0.0)
            return c_lo, c_hi

        def body(j, c, wk=wk, rwk=rwk):
            lvec, c_lo, c_hi = c
            for _ in range(_UNROLL):
                c_lo, c_hi = chunk(lvec, c_lo, c_hi, wk=wk, rwk=rwk)
                lvec = lvec + float(_LANES)
            return (lvec, c_lo, c_hi)

        _, c_lo, c_hi = lax.fori_loop(0, n_chunks // _UNROLL, body,
                                      (l_base + iota_f, zf, zf))
        blo_w = blo_w + c_lo * rwk
        blo_h = blo_h + c_lo
        bhi_w = bhi_w + c_hi * rwk
        bhi_h = bhi_h + c_hi

    col_lo = iota_i + L
    col_hi = iota_i + (L + 128)
    plsc.addupdate_scatter(wm_v, [col_lo], blo_w)
    plsc.addupdate_scatter(hc_v, [col_lo], blo_h)
    plsc.addupdate_scatter(wm_v, [col_hi], bhi_w)
    plsc.addupdate_scatter(hc_v, [col_hi], bhi_h)

    pltpu.sync_copy(wm_v, out_hbm.at[wid, 0])
    pltpu.sync_copy(hc_v, out_hbm.at[wid, 1])


def _sc_scatter(L, d16, w16, rw16):
    Lp = L + _PAD
    mesh = plsc.VectorSubcoreMesh(core_axis_name="c", subcore_axis_name="s")
    fn = functools.partial(
        pl.kernel,
        out_type=jax.ShapeDtypeStruct((_NC * _NS, 2, Lp), jnp.float32),
        mesh=mesh,
        compiler_params=pltpu.CompilerParams(needs_layout_passes=False),
        scratch_types=[
            pltpu.VMEM((_LANES,), jnp.float32),
            pltpu.VMEM((_K, _LANES), jnp.float32),
            pltpu.VMEM((_K, _LANES), jnp.float32),
            pltpu.VMEM((Lp,), jnp.float32),
            pltpu.VMEM((Lp,), jnp.float32),
        ],
    )(functools.partial(_sc_scatter_body, L))
    return fn(d16, w16, rw16)


def _tc_norm_mul_body(L, parts_ref, x_ref, o_ref, norm_ref):
    @pl.when(pl.program_id(0) == 0)
    def _():
        s = jnp.sum(parts_ref[...], axis=0)
        pw = s[0:1, :]
        ph = s[1:2, :]
        b0w = jnp.sum(pw[:, L:L + 16])
        b0h = jnp.sum(ph[:, L:L + 16])
        bLw = jnp.sum(pw[:, L + 128:L + 144])
        bLh = jnp.sum(ph[:, L + 128:L + 144])
        col = lax.broadcasted_iota(jnp.int32, (1, L), 1)
        wm = pw[:, :L] + jnp.where(col == 0, b0w, 0.0) + jnp.where(col == L - 1, bLw, 0.0)
        hc = ph[:, :L] + jnp.where(col == 0, b0h, 0.0) + jnp.where(col == L - 1, bLh, 0.0)
        avg = wm / jnp.maximum(hc, 1e-6)
        norm_ref[...] = avg / jnp.maximum(jnp.sum(avg, axis=1, keepdims=True), 1e-6)

    o_ref[...] = x_ref[...] * norm_ref[...]


def _tc_norm_mul(x, parts):
    B, L = x.shape
    Lp = L + _PAD
    rows = 32
    grid = (B // rows,)
    return pl.pallas_call(
        functools.partial(_tc_norm_mul_body, L),
        grid=grid,
        in_specs=[
            pl.BlockSpec((_NC * _NS, 2, Lp), lambda i: (0, 0, 0)),
            pl.BlockSpec((rows, L), lambda i: (i, 0)),
        ],
        out_specs=pl.BlockSpec((rows, L), lambda i: (i, 0)),
        out_shape=jax.ShapeDtypeStruct((B, L), x.dtype),
        scratch_shapes=[pltpu.VMEM((1, L), jnp.float32)],
    )(parts, x)


def kernel(x, base_deformation, base_stride, base_beta_fwd, base_beta_bwd,
           base_strength, base_alpha_fwd, base_alpha_bwd, sample_bias):
    B, L = x.shape
    K = _K
    d = jnp.clip(base_deformation, -32.0, 32.0)
    stride = jax.nn.softplus(base_stride)
    beta_fwd = jax.nn.softplus(base_beta_fwd)
    beta_bwd = jax.nn.softplus(base_beta_bwd)
    strength = jax.nn.softplus(base_strength)
    alpha_fwd = jax.nn.softplus(base_alpha_fwd)
    alpha_bwd = jax.nn.softplus(base_alpha_bwd)
    k2 = lax.broadcasted_iota(jnp.float32, (K, _LANES), 0) - (K // 2)
    k_abs = jnp.abs(k2)
    w16 = jnp.where(k2 >= 0, (k_abs ** beta_fwd) * stride,
                    -((k_abs ** beta_bwd) * stride))
    envelope = jnp.where(k2 >= 0, strength / (1.0 + k_abs) ** alpha_fwd,
                         strength / (1.0 + k_abs) ** alpha_bwd)
    bias2 = lax.broadcast_in_dim(sample_bias, (K, _LANES), (0,))
    rw16 = envelope * (1.0 + jnp.tanh(bias2))
    d16 = lax.broadcast_in_dim(d, (_LANES,), ())

    parts = _sc_scatter(L, d16, w16, rw16)
    return _tc_norm_mul(x, parts)
